# baseline (speedup 1.0000x reference)
import functools

import jax
import jax.numpy as jnp
from jax import lax
from jax.experimental import pallas as pl
from jax.experimental.pallas import tpu as pltpu
from jax.experimental.pallas import tpu_sc as plsc

_B = 8
_C = 5
_E = 100
_EP = 112
_FULL = 96


@functools.partial(
    pl.kernel,
    mesh=plsc.VectorSubcoreMesh(core_axis_name="c", subcore_axis_name="s"),
    out_type=jax.ShapeDtypeStruct((_B, 16), jnp.float32),
    scratch_types=[
        pltpu.VMEM((_C, _EP), jnp.float32),
        pltpu.VMEM((16,), jnp.float32),
        pltpu.SemaphoreType.DMA,
    ],
)
def _sc_mean(in1_hbm, out_hbm, buf_in, buf_out, sem):
    cid = lax.axis_index("c")
    sid = lax.axis_index("s")

    @pl.when((cid == 0) & (sid < _B))
    def _():
        b = sid
        copies = [
            pltpu.async_copy(in1_hbm.at[b, c, pl.ds(0, _EP)], buf_in.at[c], sem)
            for c in range(_C)
        ]
        for cp in copies:
            cp.wait()
        lane = lax.iota(jnp.int32, 16)
        outv = jnp.zeros((16,), jnp.float32)
        for c in range(_C):
            acc = buf_in[c, pl.ds(0, 16)]
            for j in range(1, _FULL // 16):
                acc = acc + buf_in[c, pl.ds(j * 16, 16)]
            tail = buf_in[c, pl.ds(_FULL, 16)]
            total = acc[0]
            for l in range(1, 16):
                total = total + acc[l]
            for e in range(_E - _FULL):
                total = total + tail[e]
            outv = jnp.where(lane == c, total * jnp.float32(1.0 / _E), outv)
        buf_out[...] = outv
        pltpu.sync_copy(buf_out, out_hbm.at[b])


def kernel(in1, in2, operation, attr1=1, attr2=1, attr3=1):
    del in2, operation, attr1, attr2, attr3
    return _sc_mean(in1)[:, :_C]
